# baseline (speedup 1.0000x reference)
import functools

import jax
import jax.numpy as jnp
from jax import lax
from jax.experimental import pallas as pl
from jax.experimental.pallas import tpu as pltpu
from jax.experimental.pallas import tpu_sc as plsc

N_NODES = 10000
D_FEAT = 128
N_EDGES = 320000
MIN_NORM = 1e-15
EPS = 4e-3

NC = 2
NS = 16
NW = NC * NS
K = 80
RB = 125
NB = 129
NBT = 136
RPT = 632
RPT_LAST = N_NODES - RPT * (NS - 1)
DV = D_FEAT // 16
GPB = K // 16

assert NB % 16 == 1 and RB * K * NW == N_EDGES and RPT_LAST % 8 == 0
assert NB >= RB and NBT % 8 == 0 and NBT >= NB + 7


def _spmm_body(table, srcb, dstb, wb, out, accum, *scr):
    sgs = scr[0:2]
    dgs = scr[2:4]
    wgs = scr[4:6]
    sbs = scr[6:10]
    dbs = scr[10:14]
    rbs = scr[14:18]
    gs = scr[18:22]
    ss = scr[22:26]
    ts = scr[26:28]
    c = lax.axis_index("c")
    s = lax.axis_index("s")
    wid = c * NS + s
    base0 = wid * NBT
    r0 = s * RPT

    def stage_group(x, m):
        r = base0 + 8 * x
        pltpu.async_copy(srcb.at[pl.ds(r, 8)], sgs[m], ts[m])
        pltpu.async_copy(dstb.at[pl.ds(r, 8)], dgs[m], ts[m])
        pltpu.async_copy(wb.at[pl.ds(r, 8)], wgs[m], ts[m])

    def stwait_group(m):
        pltpu.make_async_copy(srcb.at[pl.ds(base0, 8)], sgs[m], ts[m]).wait()
        pltpu.make_async_copy(dstb.at[pl.ds(base0, 8)], dgs[m], ts[m]).wait()
        pltpu.make_async_copy(wb.at[pl.ds(base0, 8)], wgs[m], ts[m]).wait()

    def _vcopy(dst, src2d, b8):
        for v in range(K // 16):
            sl = pl.ds(v * 16, 16)
            dst[sl] = src2d[b8, sl]

    def gather(b8, m, j):
        _vcopy(sbs[j], sgs[m], b8)
        pltpu.async_copy(table.at[sbs[j]], rbs[j], gs[j])

    def gwait(b8, m, j):
        pltpu.make_async_copy(table.at[sbs[j]], rbs[j], gs[j]).wait()

    def scatter(b8, m, j):
        _vcopy(dbs[j], dgs[m], b8)
        pltpu.async_copy(rbs[j], accum.at[dbs[j]], ss[j], add=True)

    def swait(b8, m, j):
        pltpu.make_async_copy(rbs[j], accum.at[dbs[j]], ss[j]).wait()

    def scale(b8, m, j):
        wg, rows = wgs[m], rbs[j]

        def body(g, inner):
            w16 = wg[b8, pl.ds(g * 16, 16)]
            for l in range(16):
                wl = w16[l]
                for d in range(DV):
                    sl = pl.ds(d * 16, 16)
                    rows[g * 16 + l, sl] = rows[g * 16 + l, sl] * wl
            return inner

        lax.fori_loop(0, GPB, body, 0)

    def trip(b):
        return (b % 8, (b // 8) % 2, b % 4)

    stage_group(0, 0)
    stwait_group(0)
    gather(*trip(0))
    gather(*trip(1))

    zvec = jnp.zeros((16,), jnp.float32)
    zb = rbs[3]

    def zero_row(i, carry):
        for d in range(DV):
            zb[i, pl.ds(d * 16, 16)] = zvec
        return carry

    lax.fori_loop(0, K, zero_row, 0)

    @pl.when(s < NS - 1)
    def _():
        for j in range(RPT // K):
            pltpu.sync_copy(zb, accum.at[pl.ds(r0 + j * K, K)])
        pltpu.sync_copy(zb.at[pl.ds(0, RPT % K)],
                        accum.at[pl.ds(r0 + (RPT // K) * K, RPT % K)])

    @pl.when(s == NS - 1)
    def _():
        for j in range(RPT_LAST // K):
            pltpu.sync_copy(zb, accum.at[pl.ds(r0 + j * K, K)])
        pltpu.sync_copy(zb.at[pl.ds(0, RPT_LAST % K)],
                        accum.at[pl.ds(r0 + (RPT_LAST // K) * K,
                                       RPT_LAST % K)])

    plsc.subcore_barrier()

    def step(b, x, prev, nxt2, stg, stw):
        gwait(*x)
        scale(*x)
        scatter(*x)
        if prev is not None:
            swait(*prev)
        if stg is not None:
            stage_group(*stg)
        if stw is not None:
            stwait_group(stw)
        gather(*nxt2)

    step(0, trip(0), None, trip(2), None, None)

    def sixteen(i, carry):
        b = 16 * i + 1
        g0 = 2 * i
        step(b + 0, trip(1), trip(0), trip(3), None, None)
        step(b + 1, trip(2), trip(1), trip(4), (g0 + 1, 1), None)
        step(b + 2, trip(3), trip(2), trip(5), None, None)
        step(b + 3, trip(4), trip(3), trip(6), None, None)
        step(b + 4, trip(5), trip(4), trip(7), None, None)
        step(b + 5, trip(6), trip(5), trip(8), None, 1)
        step(b + 6, trip(7), trip(6), trip(9), None, None)
        step(b + 7, trip(8), trip(7), trip(10), None, None)
        step(b + 8, trip(9), trip(8), trip(11), None, None)
        step(b + 9, trip(10), trip(9), trip(12), (g0 + 2, 0), None)
        step(b + 10, trip(11), trip(10), trip(13), None, None)
        step(b + 11, trip(12), trip(11), trip(14), None, None)
        step(b + 12, trip(13), trip(12), trip(15), None, None)
        step(b + 13, trip(14), trip(13), trip(16), None, 0)
        step(b + 14, trip(15), trip(14), trip(17), None, None)
        step(b + 15, trip(16), trip(15), trip(18), None, None)
        return carry

    lax.fori_loop(0, (NB - 1) // 16, sixteen, 0)

    gwait(*trip(NB))
    gwait(*trip(NB + 1))
    swait(*trip(NB - 1))
    plsc.subcore_barrier()

    @pl.when(s < NS - 1)
    def _():
        pltpu.sync_copy(accum.at[pl.ds(r0, RPT)], out.at[c, pl.ds(r0, RPT)])

    @pl.when(s == NS - 1)
    def _():
        pltpu.sync_copy(accum.at[pl.ds(r0, RPT_LAST)],
                        out.at[c, pl.ds(r0, RPT_LAST)])


@functools.cache
def _make_spmm():
    return pl.kernel(
        _spmm_body,
        out_type=jax.ShapeDtypeStruct((NC, N_NODES, D_FEAT), jnp.float32),
        mesh=plsc.VectorSubcoreMesh(core_axis_name="c", subcore_axis_name="s",
                                    num_cores=NC, num_subcores=NS),
        scratch_types=(
            [pltpu.VMEM_SHARED((N_NODES, D_FEAT), jnp.float32)]
            + [pltpu.VMEM((8, K), jnp.int32) for _ in range(4)]
            + [pltpu.VMEM((8, K), jnp.float32) for _ in range(2)]
            + [pltpu.VMEM((K,), jnp.int32) for _ in range(8)]
            + [pltpu.VMEM((K, D_FEAT), jnp.float32) for _ in range(4)]
            + [pltpu.SemaphoreType.DMA for _ in range(10)]
        ),
    )


def _spmm(table, srcb, dstb, wb):
    return _make_spmm()(table, srcb, dstb, wb)


def _logmap0_body(x_ref, o_ref):
    x = x_ref[...]
    norm = jnp.maximum(jnp.sqrt(jnp.sum(x * x, axis=1, keepdims=True)),
                       MIN_NORM)
    z = jnp.clip(norm, -1 + 1e-7, 1 - 1e-7)
    o_ref[...] = (0.5 * jnp.log((1 + z) / (1 - z)) / norm) * x


def _combine_body(p_ref, o_ref):
    o_ref[...] = p_ref[0] + p_ref[1]


def _finish_body(p_ref, o_ref):
    u = p_ref[0] + p_ref[1]
    un = jnp.maximum(jnp.sqrt(jnp.sum(u * u, axis=1, keepdims=True)), MIN_NORM)
    g = jnp.tanh(un) * u / un
    gn = jnp.maximum(jnp.sqrt(jnp.sum(g * g, axis=1, keepdims=True)), MIN_NORM)
    maxnorm = 1.0 - EPS
    o_ref[...] = jnp.where(gn > maxnorm, g / gn * maxnorm, g)


_BR = 1000
_row_spec = pl.BlockSpec((_BR, D_FEAT), lambda i: (i, 0))
_pair_spec = pl.BlockSpec((NC, _BR, D_FEAT), lambda i: (0, i, 0))
_row_shape = jax.ShapeDtypeStruct((N_NODES, D_FEAT), jnp.float32)

_logmap0 = pl.pallas_call(
    _logmap0_body, grid=(N_NODES // _BR,),
    in_specs=[_row_spec], out_specs=_row_spec, out_shape=_row_shape)

_combine = pl.pallas_call(
    _combine_body, grid=(N_NODES // _BR,),
    in_specs=[_pair_spec], out_specs=_row_spec, out_shape=_row_shape)

_finish = pl.pallas_call(
    _finish_body, grid=(N_NODES // _BR,),
    in_specs=[_pair_spec], out_specs=_row_spec, out_shape=_row_shape)


def kernel(x, edge_index, edge_weight):
    def _blocks(a):
        return jnp.pad(a.reshape(NW, RB, K),
                       ((0, 0), (0, NBT - RB), (0, 0))).reshape(NW * NBT, K)

    srcb = _blocks(edge_index[0].astype(jnp.int32))
    dstb = _blocks(edge_index[1].astype(jnp.int32))
    wb = _blocks(edge_weight.astype(jnp.float32))

    t = _logmap0(x)
    p1 = _spmm(t, srcb, dstb, wb)
    y1 = _combine(p1)
    p2 = _spmm(y1, srcb, dstb, wb)
    return _finish(p2)


# speedup vs baseline: 4.8381x; 4.8381x over previous
import functools

import jax
import jax.numpy as jnp
from jax import lax
from jax.experimental import pallas as pl
from jax.experimental.pallas import tpu as pltpu
from jax.experimental.pallas import tpu_sc as plsc

N_NODES = 10000
D_FEAT = 128
N_EDGES = 320000
MIN_NORM = 1e-15
EPS = 4e-3

NC = 2
NS = 16
NW = NC * NS
K = 80
NB = 125
NBR = NW * NB
RPT = 632
RPT_LAST = N_NODES - RPT * (NS - 1)
DV = D_FEAT // 16
GPB = K // 16

assert NB % 4 == 1 and NB * K * NW == N_EDGES and RPT_LAST % 8 == 0


def _spmm_body(table, srcb, dstb, wb, out, accum, *scr):
    sbs = scr[0:4]
    dbs = scr[4:8]
    wbs = scr[8:12]
    rbs = scr[12:16]
    gs = scr[16:20]
    ss = scr[20:24]
    ts = scr[24:28]
    c = lax.axis_index("c")
    s = lax.axis_index("s")
    wid = c * NS + s
    base0 = wid * NB
    r0 = s * RPT

    def stage(b, j):
        r = jnp.minimum(base0 + b, NBR - 1)
        pltpu.async_copy(srcb.at[r], sbs[j], ts[j])
        pltpu.async_copy(dstb.at[r], dbs[j], ts[j])
        pltpu.async_copy(wb.at[r], wbs[j], ts[j])

    def stwait(j):
        pltpu.make_async_copy(srcb.at[base0], sbs[j], ts[j]).wait()
        pltpu.make_async_copy(dstb.at[base0], dbs[j], ts[j]).wait()
        pltpu.make_async_copy(wb.at[base0], wbs[j], ts[j]).wait()

    def gather(j):
        pltpu.async_copy(table.at[sbs[j]], rbs[j], gs[j])

    def gwait(j):
        pltpu.make_async_copy(table.at[sbs[j]], rbs[j], gs[j]).wait()

    def scatter(j):
        pltpu.async_copy(rbs[j], accum.at[dbs[j]], ss[j], add=True)

    def swait(j):
        pltpu.make_async_copy(rbs[j], accum.at[dbs[j]], ss[j]).wait()

    def scale(j):
        wv, rows = wbs[j], rbs[j]

        def body(g, inner):
            w16 = wv[pl.ds(g * 16, 16)]
            for l in range(16):
                wl = w16[l]
                for d in range(DV):
                    sl = pl.ds(d * 16, 16)
                    rows[g * 16 + l, sl] = rows[g * 16 + l, sl] * wl
            return inner

        lax.fori_loop(0, GPB, body, 0)

    stage(0, 0)
    stage(1, 1)
    stage(2, 2)
    stwait(0)
    gather(0)
    stwait(1)
    gather(1)

    zvec = jnp.zeros((16,), jnp.float32)
    zb = rbs[3]

    def zero_row(i, carry):
        for d in range(DV):
            zb[i, pl.ds(d * 16, 16)] = zvec
        return carry

    lax.fori_loop(0, K, zero_row, 0)

    @pl.when(s < NS - 1)
    def _():
        for j in range(RPT // K):
            pltpu.sync_copy(zb, accum.at[pl.ds(r0 + j * K, K)])
        pltpu.sync_copy(zb.at[pl.ds(0, RPT % K)],
                        accum.at[pl.ds(r0 + (RPT // K) * K, RPT % K)])

    @pl.when(s == NS - 1)
    def _():
        for j in range(RPT_LAST // K):
            pltpu.sync_copy(zb, accum.at[pl.ds(r0 + j * K, K)])
        pltpu.sync_copy(zb.at[pl.ds(0, RPT_LAST % K)],
                        accum.at[pl.ds(r0 + (RPT_LAST // K) * K,
                                       RPT_LAST % K)])

    plsc.subcore_barrier()

    def step(b, j, first):
        gwait(j)
        scale(j)
        scatter(j)
        j1 = (j + 3) % 4
        if not first:
            swait(j1)
        stage(b + 3, j1)
        j2 = (j + 2) % 4
        stwait(j2)
        gather(j2)

    step(0, 0, first=True)

    def quad(i, carry):
        b = 4 * i + 1
        step(b, 1, False)
        step(b + 1, 2, False)
        step(b + 2, 3, False)
        step(b + 3, 0, False)
        return carry

    lax.fori_loop(0, (NB - 1) // 4, quad, 0)

    gwait(NB % 4)
    gwait((NB + 1) % 4)
    swait((NB - 1) % 4)
    stwait((NB + 2) % 4)
    plsc.subcore_barrier()

    @pl.when(s < NS - 1)
    def _():
        pltpu.sync_copy(accum.at[pl.ds(r0, RPT)], out.at[c, pl.ds(r0, RPT)])

    @pl.when(s == NS - 1)
    def _():
        pltpu.sync_copy(accum.at[pl.ds(r0, RPT_LAST)],
                        out.at[c, pl.ds(r0, RPT_LAST)])


@functools.cache
def _make_spmm():
    return pl.kernel(
        _spmm_body,
        out_type=jax.ShapeDtypeStruct((NC, N_NODES, D_FEAT), jnp.float32),
        mesh=plsc.VectorSubcoreMesh(core_axis_name="c", subcore_axis_name="s",
                                    num_cores=NC, num_subcores=NS),
        scratch_types=(
            [pltpu.VMEM_SHARED((N_NODES, D_FEAT), jnp.float32)]
            + [pltpu.VMEM((K,), jnp.int32) for _ in range(8)]
            + [pltpu.VMEM((K,), jnp.float32) for _ in range(4)]
            + [pltpu.VMEM((K, D_FEAT), jnp.float32) for _ in range(4)]
            + [pltpu.SemaphoreType.DMA for _ in range(12)]
        ),
    )


def _spmm(table, srcb, dstb, wb):
    return _make_spmm()(table, srcb, dstb, wb)


def _logmap0_body(x_ref, o_ref):
    x = x_ref[...]
    norm = jnp.maximum(jnp.sqrt(jnp.sum(x * x, axis=1, keepdims=True)),
                       MIN_NORM)
    z = jnp.clip(norm, -1 + 1e-7, 1 - 1e-7)
    o_ref[...] = (0.5 * jnp.log((1 + z) / (1 - z)) / norm) * x


def _combine_body(p_ref, o_ref):
    o_ref[...] = p_ref[0] + p_ref[1]


def _finish_body(p_ref, o_ref):
    u = p_ref[0] + p_ref[1]
    un = jnp.maximum(jnp.sqrt(jnp.sum(u * u, axis=1, keepdims=True)), MIN_NORM)
    g = jnp.tanh(un) * u / un
    gn = jnp.maximum(jnp.sqrt(jnp.sum(g * g, axis=1, keepdims=True)), MIN_NORM)
    maxnorm = 1.0 - EPS
    o_ref[...] = jnp.where(gn > maxnorm, g / gn * maxnorm, g)


_BR = 1000
_row_spec = pl.BlockSpec((_BR, D_FEAT), lambda i: (i, 0))
_pair_spec = pl.BlockSpec((NC, _BR, D_FEAT), lambda i: (0, i, 0))
_row_shape = jax.ShapeDtypeStruct((N_NODES, D_FEAT), jnp.float32)

_logmap0 = pl.pallas_call(
    _logmap0_body, grid=(N_NODES // _BR,),
    in_specs=[_row_spec], out_specs=_row_spec, out_shape=_row_shape)

_combine = pl.pallas_call(
    _combine_body, grid=(N_NODES // _BR,),
    in_specs=[_pair_spec], out_specs=_row_spec, out_shape=_row_shape)

_finish = pl.pallas_call(
    _finish_body, grid=(N_NODES // _BR,),
    in_specs=[_pair_spec], out_specs=_row_spec, out_shape=_row_shape)


def kernel(x, edge_index, edge_weight):
    srcb = edge_index[0].astype(jnp.int32).reshape(NBR, K)
    dstb = edge_index[1].astype(jnp.int32).reshape(NBR, K)
    wb = edge_weight.astype(jnp.float32).reshape(NBR, K)

    t = _logmap0(x)
    p1 = _spmm(t, srcb, dstb, wb)
    y1 = _combine(p1)
    p2 = _spmm(y1, srcb, dstb, wb)
    return _finish(p2)


# speedup vs baseline: 4.9321x; 1.0194x over previous
import functools

import jax
import jax.numpy as jnp
from jax import lax
from jax.experimental import pallas as pl
from jax.experimental.pallas import tpu as pltpu
from jax.experimental.pallas import tpu_sc as plsc

N_NODES = 10000
D_FEAT = 128
N_EDGES = 320000
MIN_NORM = 1e-15
EPS = 4e-3

NC = 2
NS = 16
NW = NC * NS
K = 80
NB = 125
NBR = NW * NB
RPT = 632
RPT_LAST = N_NODES - RPT * (NS - 1)
DV = D_FEAT // 16
GPB = K // 16

assert NB % 4 == 1 and NB * K * NW == N_EDGES and RPT_LAST % 8 == 0


def _spmm_body(table, srcb, dstb, wb, out, accum, *scr):
    sbs = scr[0:4]
    dbs = scr[4:8]
    wbs = scr[8:12]
    rbs = scr[12:16]
    gs = scr[16:20]
    ss = scr[20:24]
    ts = scr[24:28]
    c = lax.axis_index("c")
    s = lax.axis_index("s")
    wid = c * NS + s
    base0 = wid * NB
    r0 = s * RPT

    def stage(b, j):
        r = jnp.minimum(base0 + b, NBR - 1)
        pltpu.async_copy(srcb.at[r], sbs[j], ts[j])
        pltpu.async_copy(dstb.at[r], dbs[j], ts[j])
        pltpu.async_copy(wb.at[r], wbs[j], ts[j])

    def stwait(j):
        pltpu.make_async_copy(srcb.at[base0], sbs[j], ts[j]).wait()
        pltpu.make_async_copy(dstb.at[base0], dbs[j], ts[j]).wait()
        pltpu.make_async_copy(wb.at[base0], wbs[j], ts[j]).wait()

    def gather(j):
        pltpu.async_copy(table.at[sbs[j]], rbs[j], gs[j])

    def gwait(j):
        pltpu.make_async_copy(table.at[sbs[j]], rbs[j], gs[j]).wait()

    def scatter(j):
        pltpu.async_copy(rbs[j], accum.at[dbs[j]], ss[j], add=True)

    def swait(j):
        pltpu.make_async_copy(rbs[j], accum.at[dbs[j]], ss[j]).wait()

    def scale(j):
        wv, rows = wbs[j], rbs[j]

        def body(g, inner):
            w16 = wv[pl.ds(g * 16, 16)]
            for l in range(16):
                wl = w16[l]
                for d in range(DV):
                    sl = pl.ds(d * 16, 16)
                    rows[g * 16 + l, sl] = rows[g * 16 + l, sl] * wl
            return inner

        lax.fori_loop(0, GPB, body, 0)

    stage(0, 0)
    stage(1, 1)
    stage(2, 2)
    stwait(0)
    gather(0)
    stwait(1)
    gather(1)

    zvec = jnp.zeros((16,), jnp.float32)
    zb = rbs[3]

    def zero_row(i, carry):
        for d in range(DV):
            zb[i, pl.ds(d * 16, 16)] = zvec
        return carry

    lax.fori_loop(0, K, zero_row, 0)

    @pl.when(s < NS - 1)
    def _():
        for j in range(RPT // K):
            pltpu.sync_copy(zb, accum.at[pl.ds(r0 + j * K, K)])
        pltpu.sync_copy(zb.at[pl.ds(0, RPT % K)],
                        accum.at[pl.ds(r0 + (RPT // K) * K, RPT % K)])

    @pl.when(s == NS - 1)
    def _():
        for j in range(RPT_LAST // K):
            pltpu.sync_copy(zb, accum.at[pl.ds(r0 + j * K, K)])
        pltpu.sync_copy(zb.at[pl.ds(0, RPT_LAST % K)],
                        accum.at[pl.ds(r0 + (RPT_LAST // K) * K,
                                       RPT_LAST % K)])

    plsc.subcore_barrier()

    def step(b, j, first):
        gwait(j)
        scale(j)
        scatter(j)
        j1 = (j + 3) % 4
        if not first:
            swait(j1)
        stage(b + 3, j1)
        j2 = (j + 2) % 4
        stwait(j2)
        gather(j2)

    step(0, 0, first=True)

    def quad(i, carry):
        b = 4 * i + 1
        step(b, 1, False)
        step(b + 1, 2, False)
        step(b + 2, 3, False)
        step(b + 3, 0, False)
        return carry

    lax.fori_loop(0, (NB - 1) // 4, quad, 0)

    gwait(NB % 4)
    gwait((NB + 1) % 4)
    swait((NB - 1) % 4)
    stwait((NB + 2) % 4)
    plsc.subcore_barrier()

    @pl.when(s < NS - 1)
    def _():
        pltpu.sync_copy(accum.at[pl.ds(r0, RPT)], out.at[c, pl.ds(r0, RPT)])

    @pl.when(s == NS - 1)
    def _():
        pltpu.sync_copy(accum.at[pl.ds(r0, RPT_LAST)],
                        out.at[c, pl.ds(r0, RPT_LAST)])


@functools.cache
def _make_spmm():
    return pl.kernel(
        _spmm_body,
        out_type=jax.ShapeDtypeStruct((NC, N_NODES, D_FEAT), jnp.float32),
        mesh=plsc.VectorSubcoreMesh(core_axis_name="c", subcore_axis_name="s",
                                    num_cores=NC, num_subcores=NS),
        scratch_types=(
            [pltpu.VMEM_SHARED((N_NODES, D_FEAT), jnp.float32)]
            + [pltpu.VMEM((K,), jnp.int32) for _ in range(8)]
            + [pltpu.VMEM((K,), jnp.float32) for _ in range(4)]
            + [pltpu.VMEM((K, D_FEAT), jnp.float32) for _ in range(4)]
            + [pltpu.SemaphoreType.DMA for _ in range(12)]
        ),
    )


def _spmm(table, srcb, dstb, wb):
    return _make_spmm()(table, srcb, dstb, wb)


def _logmap0_body(x_ref, o_ref):
    x = x_ref[...]
    norm = jnp.maximum(jnp.sqrt(jnp.sum(x * x, axis=1, keepdims=True)),
                       MIN_NORM)
    z = jnp.clip(norm, -1 + 1e-7, 1 - 1e-7)
    o_ref[...] = (0.5 * jnp.log((1 + z) / (1 - z)) / norm) * x


def _combine_body(p_ref, o_ref):
    o_ref[...] = p_ref[0] + p_ref[1]


def _finish_body(p_ref, o_ref):
    u = p_ref[0] + p_ref[1]
    un = jnp.maximum(jnp.sqrt(jnp.sum(u * u, axis=1, keepdims=True)), MIN_NORM)
    g = jnp.tanh(un) * u / un
    gn = jnp.maximum(jnp.sqrt(jnp.sum(g * g, axis=1, keepdims=True)), MIN_NORM)
    maxnorm = 1.0 - EPS
    o_ref[...] = jnp.where(gn > maxnorm, g / gn * maxnorm, g)


_BR = 2000
_row_spec = pl.BlockSpec((_BR, D_FEAT), lambda i: (i, 0))
_pair_spec = pl.BlockSpec((NC, _BR, D_FEAT), lambda i: (0, i, 0))
_row_shape = jax.ShapeDtypeStruct((N_NODES, D_FEAT), jnp.float32)

_logmap0 = pl.pallas_call(
    _logmap0_body, grid=(N_NODES // _BR,),
    in_specs=[_row_spec], out_specs=_row_spec, out_shape=_row_shape)

_combine = pl.pallas_call(
    _combine_body, grid=(N_NODES // _BR,),
    in_specs=[_pair_spec], out_specs=_row_spec, out_shape=_row_shape)

_finish = pl.pallas_call(
    _finish_body, grid=(N_NODES // _BR,),
    in_specs=[_pair_spec], out_specs=_row_spec, out_shape=_row_shape)


def kernel(x, edge_index, edge_weight):
    srcb = edge_index[0].astype(jnp.int32).reshape(NBR, K)
    dstb = edge_index[1].astype(jnp.int32).reshape(NBR, K)
    wb = edge_weight.astype(jnp.float32).reshape(NBR, K)

    t = _logmap0(x)
    p1 = _spmm(t, srcb, dstb, wb)
    y1 = _combine(p1)
    p2 = _spmm(y1, srcb, dstb, wb)
    return _finish(p2)


# speedup vs baseline: 4.9986x; 1.0135x over previous
import functools

import jax
import jax.numpy as jnp
from jax import lax
from jax.experimental import pallas as pl
from jax.experimental.pallas import tpu as pltpu
from jax.experimental.pallas import tpu_sc as plsc

N_NODES = 10000
D_FEAT = 128
N_EDGES = 320000
MIN_NORM = 1e-15
EPS = 4e-3

NC = 2
NS = 16
NW = NC * NS
K = 80
NB = 125
NBR = NW * NB
RPT = 632
RPT_LAST = N_NODES - RPT * (NS - 1)
DV = D_FEAT // 16
GPB = K // 16

assert NB % 4 == 1 and NB * K * NW == N_EDGES and RPT_LAST % 8 == 0


def _spmm_body(table, srcb, dstb, wb, out, accum, *scr):
    sbs = scr[0:4]
    dbs = scr[4:8]
    wbs = scr[8:12]
    rbs = scr[12:16]
    gs = scr[16:20]
    ss = scr[20:24]
    ts = scr[24:28]
    c = lax.axis_index("c")
    s = lax.axis_index("s")
    wid = c * NS + s
    base0 = wid * NB
    r0 = s * RPT

    def stage(b, j):
        r = jnp.minimum(base0 + b, NBR - 1)
        pltpu.async_copy(srcb.at[r], sbs[j], ts[j])
        pltpu.async_copy(dstb.at[r], dbs[j], ts[j])
        pltpu.async_copy(wb.at[r], wbs[j], ts[j])

    def stwait(j):
        pltpu.make_async_copy(srcb.at[base0], sbs[j], ts[j]).wait()
        pltpu.make_async_copy(dstb.at[base0], dbs[j], ts[j]).wait()
        pltpu.make_async_copy(wb.at[base0], wbs[j], ts[j]).wait()

    def gather(j):
        pltpu.async_copy(table.at[sbs[j]], rbs[j], gs[j])

    def gwait(j):
        pltpu.make_async_copy(table.at[sbs[j]], rbs[j], gs[j]).wait()

    def scatter(j):
        pltpu.async_copy(rbs[j], accum.at[dbs[j]], ss[j], add=True)

    def swait(j):
        pltpu.make_async_copy(rbs[j], accum.at[dbs[j]], ss[j]).wait()

    def scale(j):
        wv, rows = wbs[j], rbs[j]

        def body(g, inner):
            w16 = wv[pl.ds(g * 16, 16)]
            for l in range(16):
                wl = w16[l]
                for d in range(DV):
                    sl = pl.ds(d * 16, 16)
                    rows[g * 16 + l, sl] = rows[g * 16 + l, sl] * wl
            return inner

        lax.fori_loop(0, GPB, body, 0)

    stage(0, 0)
    stage(1, 1)
    stage(2, 2)
    stwait(0)
    gather(0)
    stwait(1)
    gather(1)

    zvec = jnp.zeros((16,), jnp.float32)
    zb = rbs[3]

    def zero_row(i, carry):
        for d in range(DV):
            zb[i, pl.ds(d * 16, 16)] = zvec
        return carry

    lax.fori_loop(0, K, zero_row, 0)

    @pl.when(s < NS - 1)
    def _():
        for j in range(RPT // K):
            pltpu.sync_copy(zb, accum.at[pl.ds(r0 + j * K, K)])
        pltpu.sync_copy(zb.at[pl.ds(0, RPT % K)],
                        accum.at[pl.ds(r0 + (RPT // K) * K, RPT % K)])

    @pl.when(s == NS - 1)
    def _():
        for j in range(RPT_LAST // K):
            pltpu.sync_copy(zb, accum.at[pl.ds(r0 + j * K, K)])
        pltpu.sync_copy(zb.at[pl.ds(0, RPT_LAST % K)],
                        accum.at[pl.ds(r0 + (RPT_LAST // K) * K,
                                       RPT_LAST % K)])

    plsc.subcore_barrier()

    def step(b, j, first):
        gwait(j)
        scale(j)
        scatter(j)
        j1 = (j + 3) % 4
        if not first:
            swait(j1)
        stage(b + 3, j1)
        j2 = (j + 2) % 4
        stwait(j2)
        gather(j2)

    step(0, 0, first=True)

    def quad(i, carry):
        b = 4 * i + 1
        step(b, 1, False)
        step(b + 1, 2, False)
        step(b + 2, 3, False)
        step(b + 3, 0, False)
        return carry

    lax.fori_loop(0, (NB - 1) // 4, quad, 0)

    gwait(NB % 4)
    gwait((NB + 1) % 4)
    swait((NB - 1) % 4)
    stwait((NB + 2) % 4)
    plsc.subcore_barrier()

    @pl.when(s < NS - 1)
    def _():
        pltpu.sync_copy(accum.at[pl.ds(r0, RPT)], out.at[c, pl.ds(r0, RPT)])

    @pl.when(s == NS - 1)
    def _():
        pltpu.sync_copy(accum.at[pl.ds(r0, RPT_LAST)],
                        out.at[c, pl.ds(r0, RPT_LAST)])


@functools.cache
def _make_spmm():
    return pl.kernel(
        _spmm_body,
        out_type=jax.ShapeDtypeStruct((NC, N_NODES, D_FEAT), jnp.float32),
        mesh=plsc.VectorSubcoreMesh(core_axis_name="c", subcore_axis_name="s",
                                    num_cores=NC, num_subcores=NS),
        scratch_types=(
            [pltpu.VMEM_SHARED((N_NODES, D_FEAT), jnp.float32)]
            + [pltpu.VMEM((K,), jnp.int32) for _ in range(8)]
            + [pltpu.VMEM((K,), jnp.float32) for _ in range(4)]
            + [pltpu.VMEM((K, D_FEAT), jnp.float32) for _ in range(4)]
            + [pltpu.SemaphoreType.DMA for _ in range(12)]
        ),
    )


def _spmm(table, srcb, dstb, wb):
    return _make_spmm()(table, srcb, dstb, wb)


def _logmap0_body(x_ref, o_ref):
    x = x_ref[...]
    norm = jnp.maximum(jnp.sqrt(jnp.sum(x * x, axis=1, keepdims=True)),
                       MIN_NORM)
    z = jnp.clip(norm, -1 + 1e-7, 1 - 1e-7)
    o_ref[...] = (0.5 * jnp.log((1 + z) / (1 - z)) / norm) * x


def _combine_body(p_ref, o_ref):
    o_ref[...] = p_ref[0] + p_ref[1]


def _finish_body(p_ref, o_ref):
    u = p_ref[0] + p_ref[1]
    un = jnp.maximum(jnp.sqrt(jnp.sum(u * u, axis=1, keepdims=True)), MIN_NORM)
    g = jnp.tanh(un) * u / un
    gn = jnp.maximum(jnp.sqrt(jnp.sum(g * g, axis=1, keepdims=True)), MIN_NORM)
    maxnorm = 1.0 - EPS
    o_ref[...] = jnp.where(gn > maxnorm, g / gn * maxnorm, g)


_BR = 5000
_row_spec = pl.BlockSpec((_BR, D_FEAT), lambda i: (i, 0))
_pair_spec = pl.BlockSpec((NC, _BR, D_FEAT), lambda i: (0, i, 0))
_row_shape = jax.ShapeDtypeStruct((N_NODES, D_FEAT), jnp.float32)

_logmap0 = pl.pallas_call(
    _logmap0_body, grid=(N_NODES // _BR,),
    in_specs=[_row_spec], out_specs=_row_spec, out_shape=_row_shape)

_combine = pl.pallas_call(
    _combine_body, grid=(N_NODES // _BR,),
    in_specs=[_pair_spec], out_specs=_row_spec, out_shape=_row_shape)

_finish = pl.pallas_call(
    _finish_body, grid=(N_NODES // _BR,),
    in_specs=[_pair_spec], out_specs=_row_spec, out_shape=_row_shape)


def kernel(x, edge_index, edge_weight):
    srcb = edge_index[0].astype(jnp.int32).reshape(NBR, K)
    dstb = edge_index[1].astype(jnp.int32).reshape(NBR, K)
    wb = edge_weight.astype(jnp.float32).reshape(NBR, K)

    t = _logmap0(x)
    p1 = _spmm(t, srcb, dstb, wb)
    y1 = _combine(p1)
    p2 = _spmm(y1, srcb, dstb, wb)
    return _finish(p2)
